# baseline (speedup 1.0000x reference)
import numpy as np
import jax
import jax.numpy as jnp
from jax.experimental import pallas as pl
from jax.experimental.pallas import tpu as pltpu

_JNT = np.array([0, 5, 1, 9, 13, 17, 6, 2, 10, 14, 18, 7, 3, 11, 15, 19, 8, 4, 12, 16, 20])
_PERM = (_JNT[:, None] + np.arange(3)[None, :]).flatten()

_ROWS = 16384
_COLS = 63
_KSRC = 24
_OROWS = 64
_W = 4096
_NCC = _ROWS // _W

_PSEL = np.zeros((_OROWS, _KSRC), np.float32)
_PSEL[np.arange(_COLS), _PERM] = 1.0


def _body(p_ref, x_hbm, o_hbm, z_hbm,
          x0, x1, x2, x3, o0, o1, o2, o3, z_v,
          in_sems, out_sems, z_sem):
    xb = (x0, x1, x2, x3)
    ob = (o0, o1, o2, o3)

    def _in(c):
        return pltpu.make_async_copy(
            x_hbm.at[pl.ds(0, _KSRC), pl.ds(c * _W, _W)], xb[c], in_sems.at[c]
        )

    def _out(c):
        return pltpu.make_async_copy(
            ob[c], o_hbm.at[:, pl.ds(c * _W, _W)], out_sems.at[c]
        )

    for c in range(_NCC):
        _in(c).start()
    z_v[...] = jnp.zeros_like(z_v)
    z_cp = pltpu.make_async_copy(z_v, z_hbm, z_sem)
    z_cp.start()
    for c in range(_NCC):
        _in(c).wait()
        ob[c][...] = jnp.dot(
            p_ref[...], xb[c][...], preferred_element_type=jnp.float32
        )
        _out(c).start()
    for c in range(_NCC):
        _out(c).wait()
    z_cp.wait()


def kernel(inputs):
    x_t = inputs.T
    x_t = pltpu.with_memory_space_constraint(x_t, pltpu.HBM)
    out_t, z_t = pl.pallas_call(
        _body,
        in_specs=[
            pl.BlockSpec((_OROWS, _KSRC), lambda: (0, 0)),
            pl.BlockSpec(memory_space=pltpu.HBM),
        ],
        out_specs=[
            pl.BlockSpec(memory_space=pltpu.HBM),
            pl.BlockSpec(memory_space=pltpu.HBM),
        ],
        out_shape=[
            jax.ShapeDtypeStruct((_OROWS, _ROWS), jnp.float32),
            jax.ShapeDtypeStruct((1, _ROWS), jnp.float32),
        ],
        scratch_shapes=(
            [pltpu.VMEM((_KSRC, _W), jnp.float32) for _ in range(_NCC)]
            + [pltpu.VMEM((_OROWS, _W), jnp.float32) for _ in range(_NCC)]
            + [
                pltpu.VMEM((1, _ROWS), jnp.float32),
                pltpu.SemaphoreType.DMA((_NCC,)),
                pltpu.SemaphoreType.DMA((_NCC,)),
                pltpu.SemaphoreType.DMA,
            ]
        ),
    )(jnp.asarray(_PSEL), x_t)
    return (out_t.T[:, :_COLS], z_t.T)
